# baseline (speedup 1.0000x reference)
import jax
import jax.numpy as jnp
from jax.experimental import pallas as pl
from jax.experimental.pallas import tpu as pltpu

N_KV_HEADS = 8
HEAD_DIM = 128
MAX_SEQ_LEN = 8192
Q_LEN = 16

NCHUNK = 4
HPC = N_KV_HEADS // NCHUNK


def _update_body(pos_ref, kc_ref, vc_ref, kval_ref, vval_ref, ko_ref, vo_ref,
                 bsem, rsem):
    bulk = []
    for c in range(NCHUNK):
        sl = pl.ds(c * HPC, HPC)
        bulk.append(pltpu.make_async_copy(kc_ref.at[sl], ko_ref.at[sl], bsem))
        bulk.append(pltpu.make_async_copy(vc_ref.at[sl], vo_ref.at[sl], bsem))
    for cp in bulk:
        cp.start()
    for cp in bulk:
        cp.wait()

    rows = []
    for i in range(Q_LEN):
        p = pos_ref[i]
        w = i
        for j in range(i + 1, Q_LEN):
            w = jnp.where(pos_ref[j] == p, j, w)
        rows.append(pltpu.make_async_copy(
            kval_ref.at[:, pl.ds(w, 1), :], ko_ref.at[:, pl.ds(p, 1), :], rsem))
        rows.append(pltpu.make_async_copy(
            vval_ref.at[:, pl.ds(w, 1), :], vo_ref.at[:, pl.ds(p, 1), :], rsem))
    for cp in rows:
        cp.start()
    for cp in rows:
        cp.wait()


def kernel(k_cache, v_cache, input_pos, k_val, v_val):
    kc = k_cache.reshape(N_KV_HEADS, MAX_SEQ_LEN, HEAD_DIM)
    vc = v_cache.reshape(N_KV_HEADS, MAX_SEQ_LEN, HEAD_DIM)
    kv = k_val.reshape(N_KV_HEADS, Q_LEN, HEAD_DIM)
    vv = v_val.reshape(N_KV_HEADS, Q_LEN, HEAD_DIM)
    pos = input_pos.astype(jnp.int32)

    any_spec = pl.BlockSpec(memory_space=pl.ANY)

    grid_spec = pltpu.PrefetchScalarGridSpec(
        num_scalar_prefetch=1,
        grid=(1,),
        in_specs=[any_spec, any_spec, any_spec, any_spec],
        out_specs=[any_spec, any_spec],
        scratch_shapes=[pltpu.SemaphoreType.DMA, pltpu.SemaphoreType.DMA],
    )

    ko, vo = pl.pallas_call(
        _update_body,
        grid_spec=grid_spec,
        out_shape=[
            jax.ShapeDtypeStruct(kc.shape, kc.dtype),
            jax.ShapeDtypeStruct(vc.shape, vc.dtype),
        ],
    )(pos, kc, vc, kv, vv)

    return (ko.reshape(k_cache.shape), vo.reshape(v_cache.shape))


# speedup vs baseline: 47.1839x; 47.1839x over previous
import jax
import jax.numpy as jnp
from jax.experimental import pallas as pl
from jax.experimental.pallas import tpu as pltpu

N_KV_HEADS = 8
HEAD_DIM = 128
MAX_SEQ_LEN = 8192
Q_LEN = 16

HB = 1
NHB = N_KV_HEADS // HB


def _update_body(pos_ref, kc_ref, vc_ref, kval_ref, vval_ref, ko_ref, vo_ref):
    ko_ref[...] = kc_ref[...]
    vo_ref[...] = vc_ref[...]
    for i in range(Q_LEN):
        p = pos_ref[i]
        for lh in range(HB):
            ko_ref[lh, pl.ds(p, 1), :] = kval_ref[lh, pl.ds(i, 1), :]
            vo_ref[lh, pl.ds(p, 1), :] = vval_ref[lh, pl.ds(i, 1), :]


def kernel(k_cache, v_cache, input_pos, k_val, v_val):
    kc = k_cache.reshape(N_KV_HEADS, MAX_SEQ_LEN, HEAD_DIM)
    vc = v_cache.reshape(N_KV_HEADS, MAX_SEQ_LEN, HEAD_DIM)
    kv = k_val.reshape(N_KV_HEADS, Q_LEN, HEAD_DIM)
    vv = v_val.reshape(N_KV_HEADS, Q_LEN, HEAD_DIM)
    pos = input_pos.astype(jnp.int32)

    cache_spec = pl.BlockSpec((HB, MAX_SEQ_LEN, HEAD_DIM), lambda h, pos_ref: (h, 0, 0))
    val_spec = pl.BlockSpec((HB, Q_LEN, HEAD_DIM), lambda h, pos_ref: (h, 0, 0))

    grid_spec = pltpu.PrefetchScalarGridSpec(
        num_scalar_prefetch=1,
        grid=(NHB,),
        in_specs=[cache_spec, cache_spec, val_spec, val_spec],
        out_specs=[cache_spec, cache_spec],
    )

    ko, vo = pl.pallas_call(
        _update_body,
        grid_spec=grid_spec,
        out_shape=[
            jax.ShapeDtypeStruct(kc.shape, kc.dtype),
            jax.ShapeDtypeStruct(vc.shape, vc.dtype),
        ],
        compiler_params=pltpu.CompilerParams(
            vmem_limit_bytes=100 * 1024 * 1024,
        ),
    )(pos, kc, vc, kv, vv)

    return (ko.reshape(k_cache.shape), vo.reshape(v_cache.shape))
